# baseline (speedup 1.0000x reference)
import jax
import jax.numpy as jnp
from jax import lax
from jax.experimental import pallas as pl
from jax.experimental.pallas import tpu as pltpu
from jax.experimental.pallas import tpu_sc as plsc

BATCH = 16384
HIST = 50
HP = 56
EMBED_DIM = 32
VOCAB = 1000000
B = BATCH * HIST
NC, NS = 2, 16
NW = NC * NS
B_PER_W = BATCH // NW
GB = 8
N_GROUPS = B_PER_W // GB


def _tc_identity_body(t_ref, o_ref):
    o_ref[...] = t_ref[...]


def _gather_body(xp_hbm, tbl_hbm, out_hbm, idx_v, rows_v, gsem):
    wid = lax.axis_index("s") * NC + lax.axis_index("c")
    b0 = wid * B_PER_W
    pltpu.sync_copy(xp_hbm.at[pl.ds(b0, B_PER_W), :], idx_v)

    def grp(g, c_):
        cps = []
        for i in range(GB):
            b = g * GB + i
            cps.append(
                pltpu.async_copy(
                    tbl_hbm.at[idx_v.at[b, pl.ds(0, HP)]],
                    rows_v.at[pl.ds(i * HP, HP)],
                    gsem,
                )
            )
        for cp in cps:
            cp.wait()
        for i in range(GB):
            b = g * GB + i
            pltpu.sync_copy(
                rows_v.at[pl.ds(i * HP, HIST)],
                out_hbm.at[pl.ds((b0 + b) * HIST, HIST)],
            )
        return c_

    lax.fori_loop(0, N_GROUPS, grp, None)


@jax.jit
def _run(x, table):
    fill = (
        jnp.arange(BATCH, dtype=jnp.int32)[:, None] * 53
        + jnp.arange(6, dtype=jnp.int32)[None, :] * 131
    ) % VOCAB
    xp = jnp.concatenate([x, fill, jnp.zeros((BATCH, 72), jnp.int32)], axis=1)
    tbl128 = jnp.concatenate(
        [table[0::4], table[1::4], table[2::4], table[3::4]], axis=1
    )
    tbl128 = pl.pallas_call(
        _tc_identity_body,
        grid=(125,),
        in_specs=[pl.BlockSpec((2000, 128), lambda i: (i, 0))],
        out_specs=pl.BlockSpec((2000, 128), lambda i: (i, 0)),
        out_shape=jax.ShapeDtypeStruct((VOCAB // 4, 128), jnp.float32),
    )(tbl128)
    tblL = jnp.reshape(tbl128, (VOCAB, EMBED_DIM))

    mesh = plsc.VectorSubcoreMesh(core_axis_name="c", subcore_axis_name="s")
    out = pl.kernel(
        _gather_body,
        out_type=jax.ShapeDtypeStruct((B, EMBED_DIM), jnp.float32),
        mesh=mesh,
        scratch_types=[
            pltpu.VMEM((B_PER_W, 128), jnp.int32),
            pltpu.VMEM((GB * HP, EMBED_DIM), jnp.float32),
            pltpu.SemaphoreType.DMA,
        ],
        compiler_params=pltpu.CompilerParams(use_tc_tiling_on_sc=False),
    )(xp, tblL)
    return jnp.reshape(out, (BATCH, HIST, EMBED_DIM))


def kernel(x, table):
    return _run(x.astype(jnp.int32), table)


# speedup vs baseline: 5.0940x; 5.0940x over previous
import jax
import jax.numpy as jnp
from jax import lax
from jax.experimental import pallas as pl
from jax.experimental.pallas import tpu as pltpu
from jax.experimental.pallas import tpu_sc as plsc

BATCH = 16384
HIST = 50
HP = 56
EMBED_DIM = 32
REC = 128
VOCAB = 1000000
NC, NS = 2, 16
NW = NC * NS
B_PER_W = BATCH // NW
GB = 4
N_GROUPS = B_PER_W // GB
PAIRS = N_GROUPS // 2


def _gather_body(xp_hbm, tblp_hbm, out_hbm, idx_v, rows_a, rows_b, sem_a, sem_b):
    wid = lax.axis_index("s") * NC + lax.axis_index("c")
    b0 = wid * B_PER_W
    pltpu.sync_copy(xp_hbm.at[pl.ds(b0, B_PER_W), :], idx_v)

    def fire(g, rows, sem):
        for i in range(GB):
            b = g * GB + i
            pltpu.async_copy(
                tblp_hbm.at[idx_v.at[b, pl.ds(0, HP)]],
                rows.at[i],
                sem,
            )

    def wait_gathers(rows, sem):
        for i in range(GB):
            pltpu.make_async_copy(
                tblp_hbm.at[pl.ds(0, HP)],
                rows.at[i],
                sem,
            ).wait()

    def store(g, rows):
        pltpu.sync_copy(
            rows.at[:, pl.ds(0, HIST), pl.ds(0, EMBED_DIM)],
            out_hbm.at[pl.ds(b0 + g * GB, GB), :, :],
        )

    fire(0, rows_a, sem_a)

    def pair(p, carry):
        g0 = p * 2
        fire(g0 + 1, rows_b, sem_b)
        wait_gathers(rows_a, sem_a)
        store(g0, rows_a)

        @pl.when(g0 + 2 < N_GROUPS)
        def _():
            fire(g0 + 2, rows_a, sem_a)

        wait_gathers(rows_b, sem_b)
        store(g0 + 1, rows_b)
        return carry

    lax.fori_loop(0, PAIRS, pair, None)


@jax.jit
def _run(x, table):
    fill = (
        jnp.arange(BATCH, dtype=jnp.int32)[:, None] * 53
        + jnp.arange(6, dtype=jnp.int32)[None, :] * 131
    ) % VOCAB
    xp = jnp.concatenate([x, fill, jnp.zeros((BATCH, 72), jnp.int32)], axis=1)
    tblp = jnp.concatenate(
        [table, jnp.zeros((VOCAB, REC - EMBED_DIM), jnp.float32)], axis=1
    )

    mesh = plsc.VectorSubcoreMesh(core_axis_name="c", subcore_axis_name="s")
    return pl.kernel(
        _gather_body,
        out_type=jax.ShapeDtypeStruct((BATCH, HIST, EMBED_DIM), jnp.float32),
        mesh=mesh,
        scratch_types=[
            pltpu.VMEM((B_PER_W, 128), jnp.int32),
            pltpu.VMEM((GB, HP, REC), jnp.float32),
            pltpu.VMEM((GB, HP, REC), jnp.float32),
            pltpu.SemaphoreType.DMA,
            pltpu.SemaphoreType.DMA,
        ],
        compiler_params=pltpu.CompilerParams(use_tc_tiling_on_sc=False),
    )(xp, tblp)


def kernel(x, table):
    return _run(x.astype(jnp.int32), table)
